# baseline (speedup 1.0000x reference)
import functools

import jax
import jax.numpy as jnp
from jax import lax
from jax.experimental import pallas as pl
from jax.experimental.pallas import tpu as pltpu
from jax.experimental.pallas import tpu_sc as plsc

B = 4096
L = 50
VOCAB = 1000
VP = 1024
NLABELS = 1000
EMB = 256
HDIM = 1024
LDIM = 128

NC = 2
NS = 16
NW = NC * NS
LANES = 16
BW = B // NW
RG = 16
NCHUNK = BW // RG

_mesh = plsc.VectorSubcoreMesh(core_axis_name="c", subcore_axis_name="s")


@functools.partial(
    pl.kernel,
    out_type=[
        jax.ShapeDtypeStruct((B, VP), jnp.float32),
        jax.ShapeDtypeStruct((B, EMB), jnp.float32),
        jax.ShapeDtypeStruct((B, LDIM), jnp.float32),
    ],
    mesh=_mesh,
    compiler_params=pltpu.CompilerParams(needs_layout_passes=False),
    scratch_types=[
        pltpu.VMEM((BW, L), jnp.int32),
        pltpu.VMEM((BW,), jnp.int32),
        pltpu.VMEM((BW, EMB), jnp.float32),
        pltpu.VMEM((BW, LDIM), jnp.float32),
        pltpu.VMEM((RG, VP), jnp.float32),
        pltpu.VMEM((RG, VP), jnp.float32),
        pltpu.SemaphoreType.DMA,
        pltpu.SemaphoreType.DMA,
        pltpu.SemaphoreType.DMA,
        pltpu.SemaphoreType.DMA,
    ],
)
def _sc_lookups(insmi_hbm, inlbl_hbm, lbl_emb_hbm, prop_emb_hbm,
                counts_hbm, le_hbm, prop_hbm,
                smi_v, idx_v, lrows_v, prows_v, cnt_a, cnt_b,
                sem_le, sem_pr, sem_ca, sem_cb):
    wid = lax.axis_index("s") * NC + lax.axis_index("c")
    base = wid * BW

    pltpu.sync_copy(inlbl_hbm.at[pl.ds(base, BW)], idx_v)
    cp_le = pltpu.async_copy(lbl_emb_hbm.at[idx_v], lrows_v, sem_le)
    cp_pr = pltpu.async_copy(prop_emb_hbm.at[idx_v], prows_v, sem_pr)
    pltpu.sync_copy(insmi_hbm.at[pl.ds(base, BW)], smi_v)

    rows16 = lax.iota(jnp.int32, LANES)
    ones = jnp.full((LANES,), 1.0, jnp.float32)
    zeros = jnp.zeros((LANES,), jnp.float32)

    bufs = (cnt_a, cnt_b)
    sems = (sem_ca, sem_cb)

    for buf in bufs:
        flat = RG * VP // LANES
        UNROLL = 16
        def _zero(j, _, buf=buf):
            for u in range(UNROLL):
                buf[(j * UNROLL + u) // (VP // LANES),
                    pl.ds(((j * UNROLL + u) % (VP // LANES)) * LANES, LANES)] = zeros
            return 0
        lax.fori_loop(0, flat // UNROLL, _zero, 0)

    pending = [None, None]
    for g in range(NCHUNK):
        buf = bufs[g % 2]
        sem = sems[g % 2]
        if pending[g % 2] is not None:
            prev_g, cp = pending[g % 2]
            cp.wait()
            prow = prev_g * RG + rows16
            for l in range(L):
                tok = plsc.load_gather(smi_v, [prow, jnp.full((LANES,), l, jnp.int32)])
                plsc.store_scatter(buf, [rows16, tok], zeros)
        row = g * RG + rows16
        for l in range(L):
            tok = plsc.load_gather(smi_v, [row, jnp.full((LANES,), l, jnp.int32)])
            plsc.addupdate_scatter(buf, [rows16, tok], ones)
        pending[g % 2] = (g, pltpu.async_copy(
            buf, counts_hbm.at[pl.ds(base + g * RG, RG)], sem))

    pending[0][1].wait()
    pending[1][1].wait()

    cp_le.wait()
    pltpu.sync_copy(lrows_v, le_hbm.at[pl.ds(base, BW)])
    cp_pr.wait()
    pltpu.sync_copy(prows_v, prop_hbm.at[pl.ds(base, BW)])


BB = 512


def _tc_mlp(cnt_ref, le_ref, tok_ref, w1_ref, b1_ref, wm_ref, bm_ref,
            wv_ref, bv_ref, zm_ref, zlv_ref):
    bf = jnp.bfloat16
    h_tok = jnp.dot(cnt_ref[:, :VOCAB].astype(bf), tok_ref[...].astype(bf),
                    preferred_element_type=jnp.float32) * (1.0 / L)
    pre = (jnp.dot(h_tok.astype(bf), w1_ref[:EMB, :].astype(bf),
                   preferred_element_type=jnp.float32)
           + jnp.dot(le_ref[...].astype(bf), w1_ref[EMB:, :].astype(bf),
                     preferred_element_type=jnp.float32)
           + b1_ref[...])
    h1 = jnp.tanh(pre).astype(bf)
    zm_ref[...] = (jnp.dot(h1, wm_ref[...].astype(bf),
                           preferred_element_type=jnp.float32) + bm_ref[...])
    zlv_ref[...] = (jnp.dot(h1, wv_ref[...].astype(bf),
                            preferred_element_type=jnp.float32) + bv_ref[...])


def kernel(insmi, inlbl, inval, tok_emb, lbl_emb, W1, b1, Wm, bm, Wv, bv, prop_emb):
    insmi = insmi.astype(jnp.int32)
    inlbl = inlbl.astype(jnp.int32)

    counts, le, prop = _sc_lookups(insmi, inlbl, lbl_emb, prop_emb)

    zmean, zlogvar = pl.pallas_call(
        _tc_mlp,
        grid=(B // BB,),
        in_specs=[
            pl.BlockSpec((BB, VP), lambda i: (i, 0)),
            pl.BlockSpec((BB, EMB), lambda i: (i, 0)),
            pl.BlockSpec((VOCAB, EMB), lambda i: (0, 0)),
            pl.BlockSpec((2 * EMB, HDIM), lambda i: (0, 0)),
            pl.BlockSpec((1, HDIM), lambda i: (0, 0)),
            pl.BlockSpec((HDIM, LDIM), lambda i: (0, 0)),
            pl.BlockSpec((1, LDIM), lambda i: (0, 0)),
            pl.BlockSpec((HDIM, LDIM), lambda i: (0, 0)),
            pl.BlockSpec((1, LDIM), lambda i: (0, 0)),
        ],
        out_specs=[
            pl.BlockSpec((BB, LDIM), lambda i: (i, 0)),
            pl.BlockSpec((BB, LDIM), lambda i: (i, 0)),
        ],
        out_shape=[
            jax.ShapeDtypeStruct((B, LDIM), jnp.float32),
            jax.ShapeDtypeStruct((B, LDIM), jnp.float32),
        ],
    )(counts, le, tok_emb, W1, b1[None, :], Wm, bm[None, :], Wv, bv[None, :])

    return zmean, zlogvar, prop


# speedup vs baseline: 1.0277x; 1.0277x over previous
import functools

import jax
import jax.numpy as jnp
from jax import lax
from jax.experimental import pallas as pl
from jax.experimental.pallas import tpu as pltpu
from jax.experimental.pallas import tpu_sc as plsc

B = 4096
L = 50
VOCAB = 1000
VP = 1024
NLABELS = 1000
EMB = 256
HDIM = 1024
LDIM = 128

NC = 2
NS = 16
NW = NC * NS
LANES = 16
RG = 16

NSPLIT = 2
BH = B // NSPLIT

_mesh = plsc.VectorSubcoreMesh(core_axis_name="c", subcore_axis_name="s")


def _make_sc(Bh):
    BW = Bh // NW
    NCHUNK = BW // RG

    @functools.partial(
        pl.kernel,
        out_type=[
            jax.ShapeDtypeStruct((Bh, VP), jnp.float32),
            jax.ShapeDtypeStruct((Bh, EMB), jnp.float32),
            jax.ShapeDtypeStruct((Bh, LDIM), jnp.float32),
        ],
        mesh=_mesh,
        compiler_params=pltpu.CompilerParams(needs_layout_passes=False),
        scratch_types=[
            pltpu.VMEM((BW * L,), jnp.int32),
            pltpu.VMEM((BW,), jnp.int32),
            pltpu.VMEM((BW, EMB), jnp.float32),
            pltpu.VMEM((BW, LDIM), jnp.float32),
            pltpu.VMEM((RG, VP), jnp.float32),
            pltpu.VMEM((RG, VP), jnp.float32),
            pltpu.SemaphoreType.DMA,
            pltpu.SemaphoreType.DMA,
            pltpu.SemaphoreType.DMA,
            pltpu.SemaphoreType.DMA,
        ],
    )
    def _sc_lookups(insmi_hbm, inlbl_hbm, lbl_emb_hbm, prop_emb_hbm,
                    counts_hbm, le_hbm, prop_hbm,
                    smi_v, idx_v, lrows_v, prows_v, cnt_a, cnt_b,
                    sem_le, sem_pr, sem_ca, sem_cb):
        wid = lax.axis_index("s") * NC + lax.axis_index("c")
        base = wid * BW

        pltpu.sync_copy(inlbl_hbm.at[pl.ds(base, BW)], idx_v)
        cp_le = pltpu.async_copy(lbl_emb_hbm.at[idx_v], lrows_v, sem_le)
        cp_pr = pltpu.async_copy(prop_emb_hbm.at[idx_v], prows_v, sem_pr)
        pltpu.sync_copy(insmi_hbm.at[pl.ds(base * L, BW * L)], smi_v)

        rows16 = lax.iota(jnp.int32, LANES)
        ones = jnp.full((LANES,), 1.0, jnp.float32)
        zeros = jnp.zeros((LANES,), jnp.float32)

        bufs = (cnt_a, cnt_b)
        sems = (sem_ca, sem_cb)

        for buf in bufs:
            flat = RG * VP // LANES
            UNROLL = 16
            def _zero(j, _, buf=buf):
                for u in range(UNROLL):
                    buf[(j * UNROLL + u) // (VP // LANES),
                        pl.ds(((j * UNROLL + u) % (VP // LANES)) * LANES, LANES)] = zeros
                return 0
            lax.fori_loop(0, flat // UNROLL, _zero, 0)

        pending = [None, None]
        for g in range(NCHUNK):
            buf = bufs[g % 2]
            sem = sems[g % 2]
            if pending[g % 2] is not None:
                prev_g, cp = pending[g % 2]
                cp.wait()
                prow = (prev_g * RG + rows16) * L
                for l in range(L):
                    tok = plsc.load_gather(smi_v, [prow + l])
                    plsc.store_scatter(buf, [rows16, tok], zeros)
            row = (g * RG + rows16) * L
            for l in range(L):
                tok = plsc.load_gather(smi_v, [row + l])
                plsc.addupdate_scatter(buf, [rows16, tok], ones)
            pending[g % 2] = (g, pltpu.async_copy(
                buf, counts_hbm.at[pl.ds(base + g * RG, RG)], sem))

        for p in pending:
            if p is not None:
                p[1].wait()

        cp_le.wait()
        pltpu.sync_copy(lrows_v, le_hbm.at[pl.ds(base, BW)])
        cp_pr.wait()
        pltpu.sync_copy(prows_v, prop_hbm.at[pl.ds(base, BW)])

    return _sc_lookups


_sc_half = _make_sc(BH)

BB = 512


def _tc_mlp(cnt_ref, le_ref, tok_ref, w1_ref, b1_ref, wm_ref, bm_ref,
            wv_ref, bv_ref, zm_ref, zlv_ref):
    h_tok = jnp.dot(cnt_ref[:, :VOCAB], tok_ref[...],
                    preferred_element_type=jnp.float32) * (1.0 / L)
    pre = (jnp.dot(h_tok, w1_ref[:EMB, :], preferred_element_type=jnp.float32)
           + jnp.dot(le_ref[...], w1_ref[EMB:, :], preferred_element_type=jnp.float32)
           + b1_ref[...])
    h1 = jnp.tanh(pre)
    zm_ref[...] = (jnp.dot(h1, wm_ref[...], preferred_element_type=jnp.float32)
                   + bm_ref[...])
    zlv_ref[...] = (jnp.dot(h1, wv_ref[...], preferred_element_type=jnp.float32)
                    + bv_ref[...])


def _tc_call(counts, le, tok_emb, W1, b1, Wm, bm, Wv, bv):
    Bh = counts.shape[0]
    return pl.pallas_call(
        _tc_mlp,
        grid=(Bh // BB,),
        in_specs=[
            pl.BlockSpec((BB, VP), lambda i: (i, 0)),
            pl.BlockSpec((BB, EMB), lambda i: (i, 0)),
            pl.BlockSpec((VOCAB, EMB), lambda i: (0, 0)),
            pl.BlockSpec((2 * EMB, HDIM), lambda i: (0, 0)),
            pl.BlockSpec((1, HDIM), lambda i: (0, 0)),
            pl.BlockSpec((HDIM, LDIM), lambda i: (0, 0)),
            pl.BlockSpec((1, LDIM), lambda i: (0, 0)),
            pl.BlockSpec((HDIM, LDIM), lambda i: (0, 0)),
            pl.BlockSpec((1, LDIM), lambda i: (0, 0)),
        ],
        out_specs=[
            pl.BlockSpec((BB, LDIM), lambda i: (i, 0)),
            pl.BlockSpec((BB, LDIM), lambda i: (i, 0)),
        ],
        out_shape=[
            jax.ShapeDtypeStruct((Bh, LDIM), jnp.float32),
            jax.ShapeDtypeStruct((Bh, LDIM), jnp.float32),
        ],
    )(counts, le, tok_emb, W1, b1[None, :], Wm, bm[None, :], Wv, bv[None, :])


def kernel(insmi, inlbl, inval, tok_emb, lbl_emb, W1, b1, Wm, bm, Wv, bv, prop_emb):
    insmi = insmi.astype(jnp.int32)
    inlbl = inlbl.astype(jnp.int32)
    smi_flat = insmi.reshape(-1)

    sc_outs = []
    for h in range(NSPLIT):
        sc_outs.append(_sc_half(
            smi_flat[h * BH * L:(h + 1) * BH * L],
            inlbl[h * BH:(h + 1) * BH],
            lbl_emb, prop_emb))

    tc_outs = [
        _tc_call(c, le, tok_emb, W1, b1, Wm, bm, Wv, bv)
        for (c, le, _p) in sc_outs
    ]

    zmean = jnp.concatenate([t[0] for t in tc_outs], axis=0)
    zlogvar = jnp.concatenate([t[1] for t in tc_outs], axis=0)
    prop = jnp.concatenate([p for (_c, _le, p) in sc_outs], axis=0)
    return zmean, zlogvar, prop


# speedup vs baseline: 1.0936x; 1.0642x over previous
import functools

import jax
import jax.numpy as jnp
from jax import lax
from jax.experimental import pallas as pl
from jax.experimental.pallas import tpu as pltpu
from jax.experimental.pallas import tpu_sc as plsc

B = 4096
L = 50
VOCAB = 1000
VP = 1024
NLABELS = 1000
EMB = 256
HDIM = 1024
LDIM = 128

NC = 2
NS = 16
NW = NC * NS
LANES = 16
BW = B // NW
RG = 16
NCHUNK = BW // RG

_mesh = plsc.VectorSubcoreMesh(core_axis_name="c", subcore_axis_name="s")


@functools.partial(
    pl.kernel,
    out_type=[
        jax.ShapeDtypeStruct((B, VP), jnp.float32),
        jax.ShapeDtypeStruct((B, EMB), jnp.float32),
        jax.ShapeDtypeStruct((B, LDIM), jnp.float32),
    ],
    mesh=_mesh,
    compiler_params=pltpu.CompilerParams(needs_layout_passes=False),
    scratch_types=[
        pltpu.VMEM((BW * L,), jnp.int32),
        pltpu.VMEM((BW,), jnp.int32),
        pltpu.VMEM((BW, EMB), jnp.float32),
        pltpu.VMEM((BW, LDIM), jnp.float32),
        pltpu.VMEM((RG, VP), jnp.float32),
        pltpu.VMEM((RG, VP), jnp.float32),
        pltpu.SemaphoreType.DMA,
        pltpu.SemaphoreType.DMA,
        pltpu.SemaphoreType.DMA,
        pltpu.SemaphoreType.DMA,
    ],
)
def _sc_lookups(insmi_hbm, inlbl_hbm, lbl_emb_hbm, prop_emb_hbm,
                counts_hbm, le_hbm, prop_hbm,
                smi_v, idx_v, lrows_v, prows_v, cnt_a, cnt_b,
                sem_le, sem_pr, sem_ca, sem_cb):
    wid = lax.axis_index("s") * NC + lax.axis_index("c")
    base = wid * BW

    pltpu.sync_copy(inlbl_hbm.at[pl.ds(base, BW)], idx_v)
    cp_le = pltpu.async_copy(lbl_emb_hbm.at[idx_v], lrows_v, sem_le)
    cp_pr = pltpu.async_copy(prop_emb_hbm.at[idx_v], prows_v, sem_pr)
    pltpu.sync_copy(insmi_hbm.at[pl.ds(base * L, BW * L)], smi_v)

    rows16 = lax.iota(jnp.int32, LANES)
    ones = jnp.full((LANES,), 1.0, jnp.float32)
    zeros = jnp.zeros((LANES,), jnp.float32)

    bufs = (cnt_a, cnt_b)
    sems = (sem_ca, sem_cb)

    for buf in bufs:
        flat = RG * VP // LANES
        UNROLL = 16
        def _zero(j, _, buf=buf):
            for u in range(UNROLL):
                buf[(j * UNROLL + u) // (VP // LANES),
                    pl.ds(((j * UNROLL + u) % (VP // LANES)) * LANES, LANES)] = zeros
            return 0
        lax.fori_loop(0, flat // UNROLL, _zero, 0)

    pending = [None, None]
    for gp in range(0, NCHUNK, 2):
        if pending[0] is not None:
            pg_a, cp_a = pending[0]
            pg_b, cp_b = pending[1]
            cp_a.wait()
            cp_b.wait()
            prow_a = (pg_a * RG + rows16) * L
            prow_b = (pg_b * RG + rows16) * L
            for l in range(L):
                tok_a = plsc.load_gather(smi_v, [prow_a + l])
                tok_b = plsc.load_gather(smi_v, [prow_b + l])
                plsc.store_scatter(cnt_a, [rows16, tok_a], zeros)
                plsc.store_scatter(cnt_b, [rows16, tok_b], zeros)
        row_a = (gp * RG + rows16) * L
        row_b = ((gp + 1) * RG + rows16) * L
        for l in range(L):
            tok_a = plsc.load_gather(smi_v, [row_a + l])
            tok_b = plsc.load_gather(smi_v, [row_b + l])
            plsc.addupdate_scatter(cnt_a, [rows16, tok_a], ones)
            plsc.addupdate_scatter(cnt_b, [rows16, tok_b], ones)
        pending[0] = (gp, pltpu.async_copy(
            cnt_a, counts_hbm.at[pl.ds(base + gp * RG, RG)], sem_ca))
        pending[1] = (gp + 1, pltpu.async_copy(
            cnt_b, counts_hbm.at[pl.ds(base + (gp + 1) * RG, RG)], sem_cb))

    pending[0][1].wait()
    pending[1][1].wait()

    cp_le.wait()
    pltpu.sync_copy(lrows_v, le_hbm.at[pl.ds(base, BW)])
    cp_pr.wait()
    pltpu.sync_copy(prows_v, prop_hbm.at[pl.ds(base, BW)])


BB = 512


def _tc_mlp(cnt_ref, le_ref, tok_ref, w1_ref, b1_ref, wm_ref, bm_ref,
            wv_ref, bv_ref, zm_ref, zlv_ref):
    h_tok = jnp.dot(cnt_ref[:, :VOCAB], tok_ref[...],
                    preferred_element_type=jnp.float32) * (1.0 / L)
    pre = (jnp.dot(h_tok, w1_ref[:EMB, :], preferred_element_type=jnp.float32)
           + jnp.dot(le_ref[...], w1_ref[EMB:, :], preferred_element_type=jnp.float32)
           + b1_ref[...])
    h1 = jnp.tanh(pre)
    zm_ref[...] = (jnp.dot(h1, wm_ref[...], preferred_element_type=jnp.float32)
                   + bm_ref[...])
    zlv_ref[...] = (jnp.dot(h1, wv_ref[...], preferred_element_type=jnp.float32)
                    + bv_ref[...])


def kernel(insmi, inlbl, inval, tok_emb, lbl_emb, W1, b1, Wm, bm, Wv, bv, prop_emb):
    insmi = insmi.astype(jnp.int32)
    inlbl = inlbl.astype(jnp.int32)

    counts, le, prop = _sc_lookups(insmi.reshape(-1), inlbl, lbl_emb, prop_emb)

    zmean, zlogvar = pl.pallas_call(
        _tc_mlp,
        grid=(B // BB,),
        in_specs=[
            pl.BlockSpec((BB, VP), lambda i: (i, 0)),
            pl.BlockSpec((BB, EMB), lambda i: (i, 0)),
            pl.BlockSpec((VOCAB, EMB), lambda i: (0, 0)),
            pl.BlockSpec((2 * EMB, HDIM), lambda i: (0, 0)),
            pl.BlockSpec((1, HDIM), lambda i: (0, 0)),
            pl.BlockSpec((HDIM, LDIM), lambda i: (0, 0)),
            pl.BlockSpec((1, LDIM), lambda i: (0, 0)),
            pl.BlockSpec((HDIM, LDIM), lambda i: (0, 0)),
            pl.BlockSpec((1, LDIM), lambda i: (0, 0)),
        ],
        out_specs=[
            pl.BlockSpec((BB, LDIM), lambda i: (i, 0)),
            pl.BlockSpec((BB, LDIM), lambda i: (i, 0)),
        ],
        out_shape=[
            jax.ShapeDtypeStruct((B, LDIM), jnp.float32),
            jax.ShapeDtypeStruct((B, LDIM), jnp.float32),
        ],
    )(counts, le, tok_emb, W1, b1[None, :], Wm, bm[None, :], Wv, bv[None, :])

    return zmean, zlogvar, prop


# speedup vs baseline: 1.1212x; 1.0252x over previous
import functools

import jax
import jax.numpy as jnp
from jax import lax
from jax.experimental import pallas as pl
from jax.experimental.pallas import tpu as pltpu
from jax.experimental.pallas import tpu_sc as plsc

B = 4096
L = 50
VOCAB = 1000
VP = 1024
NLABELS = 1000
EMB = 256
HDIM = 1024
LDIM = 128

NC = 2
NS = 16
NW = NC * NS
LANES = 16
BW = B // NW
RG = 16
NCHUNK = BW // RG

_mesh = plsc.VectorSubcoreMesh(core_axis_name="c", subcore_axis_name="s")


@functools.partial(
    pl.kernel,
    out_type=[
        jax.ShapeDtypeStruct((B, VP), jnp.float32),
        jax.ShapeDtypeStruct((B, EMB), jnp.float32),
        jax.ShapeDtypeStruct((B, LDIM), jnp.float32),
    ],
    mesh=_mesh,
    compiler_params=pltpu.CompilerParams(needs_layout_passes=False),
    scratch_types=[
        pltpu.VMEM((L, BW), jnp.int32),
        pltpu.VMEM((BW,), jnp.int32),
        pltpu.VMEM((BW, EMB), jnp.float32),
        pltpu.VMEM((BW, LDIM), jnp.float32),
        pltpu.VMEM((RG, VP), jnp.float32),
        pltpu.VMEM((RG, VP), jnp.float32),
        pltpu.SemaphoreType.DMA,
        pltpu.SemaphoreType.DMA,
        pltpu.SemaphoreType.DMA,
        pltpu.SemaphoreType.DMA,
    ],
)
def _sc_lookups(insmi_hbm, inlbl_hbm, lbl_emb_hbm, prop_emb_hbm,
                counts_hbm, le_hbm, prop_hbm,
                smi_v, idx_v, lrows_v, prows_v, cnt_a, cnt_b,
                sem_le, sem_pr, sem_ca, sem_cb):
    wid = lax.axis_index("s") * NC + lax.axis_index("c")
    base = wid * BW

    pltpu.sync_copy(inlbl_hbm.at[pl.ds(base, BW)], idx_v)
    cp_le = pltpu.async_copy(lbl_emb_hbm.at[idx_v], lrows_v, sem_le)
    cp_pr = pltpu.async_copy(prop_emb_hbm.at[idx_v], prows_v, sem_pr)
    pltpu.sync_copy(insmi_hbm.at[:, pl.ds(base, BW)], smi_v)

    rows16 = lax.iota(jnp.int32, LANES)
    ones = jnp.full((LANES,), 1.0, jnp.float32)
    zeros = jnp.zeros((LANES,), jnp.float32)

    bufs = (cnt_a, cnt_b)
    sems = (sem_ca, sem_cb)

    for buf in bufs:
        flat = RG * VP // LANES
        UNROLL = 16
        def _zero(j, _, buf=buf):
            for u in range(UNROLL):
                buf[(j * UNROLL + u) // (VP // LANES),
                    pl.ds(((j * UNROLL + u) % (VP // LANES)) * LANES, LANES)] = zeros
            return 0
        lax.fori_loop(0, flat // UNROLL, _zero, 0)

    pending = [None, None]
    for gp in range(0, NCHUNK, 2):
        if pending[0] is not None:
            pg_a, cp_a = pending[0]
            pg_b, cp_b = pending[1]
            cp_a.wait()
            cp_b.wait()
            for l in range(L):
                tok_a = smi_v[l, pl.ds(pg_a * RG, RG)]
                tok_b = smi_v[l, pl.ds(pg_b * RG, RG)]
                plsc.store_scatter(cnt_a, [rows16, tok_a], zeros)
                plsc.store_scatter(cnt_b, [rows16, tok_b], zeros)
        for l in range(L):
            tok_a = smi_v[l, pl.ds(gp * RG, RG)]
            tok_b = smi_v[l, pl.ds((gp + 1) * RG, RG)]
            plsc.addupdate_scatter(cnt_a, [rows16, tok_a], ones)
            plsc.addupdate_scatter(cnt_b, [rows16, tok_b], ones)
        pending[0] = (gp, pltpu.async_copy(
            cnt_a, counts_hbm.at[pl.ds(base + gp * RG, RG)], sem_ca))
        pending[1] = (gp + 1, pltpu.async_copy(
            cnt_b, counts_hbm.at[pl.ds(base + (gp + 1) * RG, RG)], sem_cb))

    pending[0][1].wait()
    pending[1][1].wait()

    cp_le.wait()
    pltpu.sync_copy(lrows_v, le_hbm.at[pl.ds(base, BW)])
    cp_pr.wait()
    pltpu.sync_copy(prows_v, prop_hbm.at[pl.ds(base, BW)])


BB = 512


def _tc_mlp(cnt_ref, le_ref, tok_ref, w1_ref, b1_ref, wm_ref, bm_ref,
            wv_ref, bv_ref, zm_ref, zlv_ref):
    h_tok = jnp.dot(cnt_ref[:, :VOCAB], tok_ref[...],
                    preferred_element_type=jnp.float32) * (1.0 / L)
    pre = (jnp.dot(h_tok, w1_ref[:EMB, :], preferred_element_type=jnp.float32)
           + jnp.dot(le_ref[...], w1_ref[EMB:, :], preferred_element_type=jnp.float32)
           + b1_ref[...])
    h1 = jnp.tanh(pre)
    zm_ref[...] = (jnp.dot(h1, wm_ref[...], preferred_element_type=jnp.float32)
                   + bm_ref[...])
    zlv_ref[...] = (jnp.dot(h1, wv_ref[...], preferred_element_type=jnp.float32)
                    + bv_ref[...])


def kernel(insmi, inlbl, inval, tok_emb, lbl_emb, W1, b1, Wm, bm, Wv, bv, prop_emb):
    insmi = insmi.astype(jnp.int32)
    inlbl = inlbl.astype(jnp.int32)

    counts, le, prop = _sc_lookups(insmi.T, inlbl, lbl_emb, prop_emb)

    zmean, zlogvar = pl.pallas_call(
        _tc_mlp,
        grid=(B // BB,),
        in_specs=[
            pl.BlockSpec((BB, VP), lambda i: (i, 0)),
            pl.BlockSpec((BB, EMB), lambda i: (i, 0)),
            pl.BlockSpec((VOCAB, EMB), lambda i: (0, 0)),
            pl.BlockSpec((2 * EMB, HDIM), lambda i: (0, 0)),
            pl.BlockSpec((1, HDIM), lambda i: (0, 0)),
            pl.BlockSpec((HDIM, LDIM), lambda i: (0, 0)),
            pl.BlockSpec((1, LDIM), lambda i: (0, 0)),
            pl.BlockSpec((HDIM, LDIM), lambda i: (0, 0)),
            pl.BlockSpec((1, LDIM), lambda i: (0, 0)),
        ],
        out_specs=[
            pl.BlockSpec((BB, LDIM), lambda i: (i, 0)),
            pl.BlockSpec((BB, LDIM), lambda i: (i, 0)),
        ],
        out_shape=[
            jax.ShapeDtypeStruct((B, LDIM), jnp.float32),
            jax.ShapeDtypeStruct((B, LDIM), jnp.float32),
        ],
    )(counts, le, tok_emb, W1, b1[None, :], Wm, bm[None, :], Wv, bv[None, :])

    return zmean, zlogvar, prop


# speedup vs baseline: 1.1398x; 1.0166x over previous
import functools

import jax
import jax.numpy as jnp
from jax import lax
from jax.experimental import pallas as pl
from jax.experimental.pallas import tpu as pltpu
from jax.experimental.pallas import tpu_sc as plsc

B = 4096
L = 50
VOCAB = 1000
VP = 1024
NLABELS = 1000
EMB = 256
HDIM = 1024
LDIM = 128

NC = 2
NS = 16
NW = NC * NS
LANES = 16
BW = B // NW
RG = 16
NCHUNK = BW // RG

_mesh = plsc.VectorSubcoreMesh(core_axis_name="c", subcore_axis_name="s")


@functools.partial(
    pl.kernel,
    out_type=[
        jax.ShapeDtypeStruct((B, VP), jnp.float32),
        jax.ShapeDtypeStruct((B, EMB), jnp.float32),
        jax.ShapeDtypeStruct((B, LDIM), jnp.float32),
    ],
    mesh=_mesh,
    compiler_params=pltpu.CompilerParams(needs_layout_passes=False),
    scratch_types=[
        pltpu.VMEM((L, BW), jnp.int32),
        pltpu.VMEM((BW,), jnp.int32),
        pltpu.VMEM((BW, EMB), jnp.float32),
        pltpu.VMEM((BW, LDIM), jnp.float32),
        pltpu.VMEM((RG, VP), jnp.float32),
        pltpu.VMEM((RG, VP), jnp.float32),
        pltpu.SemaphoreType.DMA,
        pltpu.SemaphoreType.DMA,
        pltpu.SemaphoreType.DMA,
        pltpu.SemaphoreType.DMA,
    ],
)
def _sc_lookups(insmi_hbm, inlbl_hbm, lbl_emb_hbm, prop_emb_hbm,
                counts_hbm, le_hbm, prop_hbm,
                smi_v, idx_v, lrows_v, prows_v, cnt_a, cnt_b,
                sem_le, sem_pr, sem_ca, sem_cb):
    wid = lax.axis_index("s") * NC + lax.axis_index("c")
    base = wid * BW

    pltpu.sync_copy(inlbl_hbm.at[pl.ds(base, BW)], idx_v)
    cp_le = pltpu.async_copy(lbl_emb_hbm.at[idx_v], lrows_v, sem_le)
    cp_pr = pltpu.async_copy(prop_emb_hbm.at[idx_v], prows_v, sem_pr)
    pltpu.sync_copy(insmi_hbm.at[:, pl.ds(base, BW)], smi_v)

    rows16 = lax.iota(jnp.int32, LANES)
    ones = jnp.full((LANES,), 1.0, jnp.float32)
    zeros = jnp.zeros((LANES,), jnp.float32)

    bufs = (cnt_a, cnt_b)
    sems = (sem_ca, sem_cb)

    for buf in bufs:
        flat = RG * VP // LANES
        UNROLL = 16
        def _zero(j, _, buf=buf):
            for u in range(UNROLL):
                buf[(j * UNROLL + u) // (VP // LANES),
                    pl.ds(((j * UNROLL + u) % (VP // LANES)) * LANES, LANES)] = zeros
            return 0
        lax.fori_loop(0, flat // UNROLL, _zero, 0)

    pending = [None, None]
    for gp in range(0, NCHUNK, 2):
        if pending[0] is not None:
            pg_a, cp_a = pending[0]
            pg_b, cp_b = pending[1]
            cp_a.wait()
            cp_b.wait()
            for l in range(L):
                tok_a = smi_v[l, pl.ds(pg_a * RG, RG)]
                tok_b = smi_v[l, pl.ds(pg_b * RG, RG)]
                plsc.store_scatter(cnt_a, [rows16, tok_a], zeros)
                plsc.store_scatter(cnt_b, [rows16, tok_b], zeros)
        for l in range(L):
            tok_a = smi_v[l, pl.ds(gp * RG, RG)]
            tok_b = smi_v[l, pl.ds((gp + 1) * RG, RG)]
            plsc.addupdate_scatter(cnt_a, [rows16, tok_a], ones)
            plsc.addupdate_scatter(cnt_b, [rows16, tok_b], ones)
        pending[0] = (gp, pltpu.async_copy(
            cnt_a, counts_hbm.at[pl.ds(base + gp * RG, RG)], sem_ca))
        pending[1] = (gp + 1, pltpu.async_copy(
            cnt_b, counts_hbm.at[pl.ds(base + (gp + 1) * RG, RG)], sem_cb))

    pending[0][1].wait()
    pending[1][1].wait()

    cp_le.wait()
    pltpu.sync_copy(lrows_v, le_hbm.at[pl.ds(base, BW)])
    cp_pr.wait()
    pltpu.sync_copy(prows_v, prop_hbm.at[pl.ds(base, BW)])


BB = 1024


def _tc_mlp(cnt_ref, le_ref, tok_ref, w1_ref, b1_ref, wm_ref, bm_ref,
            wv_ref, bv_ref, zm_ref, zlv_ref):
    h_tok = jnp.dot(cnt_ref[:, :VOCAB], tok_ref[...],
                    preferred_element_type=jnp.float32) * (1.0 / L)
    pre = (jnp.dot(h_tok, w1_ref[:EMB, :], preferred_element_type=jnp.float32)
           + jnp.dot(le_ref[...], w1_ref[EMB:, :], preferred_element_type=jnp.float32)
           + b1_ref[...])
    h1 = jnp.tanh(pre)
    zm_ref[...] = (jnp.dot(h1, wm_ref[...], preferred_element_type=jnp.float32)
                   + bm_ref[...])
    zlv_ref[...] = (jnp.dot(h1, wv_ref[...], preferred_element_type=jnp.float32)
                    + bv_ref[...])


def kernel(insmi, inlbl, inval, tok_emb, lbl_emb, W1, b1, Wm, bm, Wv, bv, prop_emb):
    insmi = insmi.astype(jnp.int32)
    inlbl = inlbl.astype(jnp.int32)

    counts, le, prop = _sc_lookups(insmi.T, inlbl, lbl_emb, prop_emb)

    zmean, zlogvar = pl.pallas_call(
        _tc_mlp,
        grid=(B // BB,),
        in_specs=[
            pl.BlockSpec((BB, VP), lambda i: (i, 0)),
            pl.BlockSpec((BB, EMB), lambda i: (i, 0)),
            pl.BlockSpec((VOCAB, EMB), lambda i: (0, 0)),
            pl.BlockSpec((2 * EMB, HDIM), lambda i: (0, 0)),
            pl.BlockSpec((1, HDIM), lambda i: (0, 0)),
            pl.BlockSpec((HDIM, LDIM), lambda i: (0, 0)),
            pl.BlockSpec((1, LDIM), lambda i: (0, 0)),
            pl.BlockSpec((HDIM, LDIM), lambda i: (0, 0)),
            pl.BlockSpec((1, LDIM), lambda i: (0, 0)),
        ],
        out_specs=[
            pl.BlockSpec((BB, LDIM), lambda i: (i, 0)),
            pl.BlockSpec((BB, LDIM), lambda i: (i, 0)),
        ],
        out_shape=[
            jax.ShapeDtypeStruct((B, LDIM), jnp.float32),
            jax.ShapeDtypeStruct((B, LDIM), jnp.float32),
        ],
    )(counts, le, tok_emb, W1, b1[None, :], Wm, bm[None, :], Wv, bv[None, :])

    return zmean, zlogvar, prop
